# baseline (speedup 1.0000x reference)
import functools

import jax
import jax.numpy as jnp
from jax import lax
from jax.experimental import pallas as pl
from jax.experimental.pallas import tpu as pltpu
from jax.experimental.pallas import tpu_sc as plsc

NKEYS = 100000
D = 128
QT = 256
KC = 1024
G1 = 8
G2 = 128
NKP = 102400
NG1 = NKP // G1
NG2 = NKP // G2
TOPK = 64
TQT = 256


def _score_body(qn_ref, kn_ref, q_ref, k_ref, s_ref, l1_ref, gm_ref):
    kc = pl.program_id(1)
    q = q_ref[...]
    k = k_ref[...]
    s = lax.dot_general(q, k, (((1,), (1,)), ((), ())),
                        preferred_element_type=jnp.float32)
    rq = 1.0 / jnp.maximum(qn_ref[...], 1e-30)
    rk = 1.0 / jnp.maximum(kn_ref[...], 1e-30)
    cos = s * rq * rk
    col = kc * KC + lax.broadcasted_iota(jnp.int32, (QT, KC), 1)
    cos = jnp.where(col < NKEYS, cos, -1e30)
    s_ref[...] = s
    l1 = cos[:, 0:G2]
    for j in range(1, KC // G2):
        l1 = jnp.maximum(l1, cos[:, j * G2:(j + 1) * G2])
    l1_ref[0] = l1
    m = l1
    for sh in (8, 4, 2, 1):
        m = jnp.maximum(m, pltpu.roll(m, G2 - sh, 1))
    nh = KC // G2
    onehot = (lax.broadcasted_iota(jnp.int32, (G2, nh), 0)
              == 16 * lax.broadcasted_iota(jnp.int32, (G2, nh), 1)
              ).astype(jnp.float32)
    gm_ref[0] = lax.dot_general(m, onehot, (((1,), (0,)), ((), ())),
                                precision=lax.Precision.HIGHEST,
                                preferred_element_type=jnp.float32)


def _scores_and_group_max(query, key_pad, qn, kn):
    B = query.shape[0]
    return pl.pallas_call(
        _score_body,
        grid=(B // QT, NKP // KC),
        in_specs=[
            pl.BlockSpec((QT, 1), lambda q, c: (q, 0)),
            pl.BlockSpec((1, KC), lambda q, c: (0, c)),
            pl.BlockSpec((QT, D), lambda q, c: (q, 0)),
            pl.BlockSpec((KC, D), lambda q, c: (c, 0)),
        ],
        out_specs=[
            pl.BlockSpec((QT, KC), lambda q, c: (q, c)),
            pl.BlockSpec((1, QT, G2), lambda q, c: (c, q, 0)),
            pl.BlockSpec((1, QT, KC // G2), lambda q, c: (c, q, 0)),
        ],
        out_shape=[
            jax.ShapeDtypeStruct((B, NKP), jnp.float32),
            jax.ShapeDtypeStruct((NKP // KC, B, G2), jnp.float32),
            jax.ShapeDtypeStruct((NKP // KC, B, KC // G2), jnp.float32),
        ],
        compiler_params=pltpu.CompilerParams(
            dimension_semantics=("parallel", "arbitrary")),
    )(qn, kn, query, key_pad)


def _topk_body(x_ref, w_ref, i_ref, xs_ref):
    R, W = x_ref.shape
    xs_ref[...] = x_ref[...]
    ii = lax.broadcasted_iota(jnp.int32, (R, W), 1)
    t64 = lax.broadcasted_iota(jnp.int32, (R, TOPK), 1)

    def body(t, carry):
        w_acc, i_acc = carry
        x = xs_ref[...]
        m = jnp.max(x, axis=1, keepdims=True)
        am = jnp.min(jnp.where(x == m, ii, W), axis=1, keepdims=True)
        xs_ref[...] = jnp.where(ii == am, -jnp.inf, x)
        w_acc = jnp.where(t64 == t, m, w_acc)
        i_acc = jnp.where(t64 == t, am, i_acc)
        return w_acc, i_acc

    w, i = lax.fori_loop(
        0, TOPK, body,
        (jnp.zeros((R, TOPK), jnp.float32), jnp.zeros((R, TOPK), jnp.int32)))
    w_ref[...] = w
    i_ref[...] = i


def _topk64(x):
    B, W = x.shape
    return pl.pallas_call(
        _topk_body,
        grid=(B // TQT,),
        in_specs=[pl.BlockSpec((TQT, W), lambda r: (r, 0))],
        out_specs=[
            pl.BlockSpec((TQT, TOPK), lambda r: (r, 0)),
            pl.BlockSpec((TQT, TOPK), lambda r: (r, 0)),
        ],
        out_shape=[
            jax.ShapeDtypeStruct((B, TOPK), jnp.float32),
            jax.ShapeDtypeStruct((B, TOPK), jnp.int32),
        ],
        scratch_shapes=[pltpu.VMEM((TQT, W), jnp.float32)],
        compiler_params=pltpu.CompilerParams(
            dimension_semantics=("parallel",)),
    )(x)


def _topk_idx_body(x_ref, n_ref, c_ref, w_ref, i_ref, xs_ref):
    R, W = x_ref.shape
    cmap = c_ref[...]
    norm = n_ref[...]
    norm = jnp.where(norm > 1e-8, norm, 1e-8)
    xs_ref[...] = jnp.where(cmap < NKEYS, x_ref[...] / norm, -jnp.inf)
    t64 = lax.broadcasted_iota(jnp.int32, (R, TOPK), 1)

    def body(t, carry):
        w_acc, i_acc = carry
        x = xs_ref[...]
        m = jnp.max(x, axis=1, keepdims=True)
        am = jnp.min(jnp.where(x == m, cmap, 2 ** 31 - 1), axis=1,
                     keepdims=True)
        xs_ref[...] = jnp.where(cmap == am, -jnp.inf, x)
        w_acc = jnp.where(t64 == t, m, w_acc)
        i_acc = jnp.where(t64 == t, am, i_acc)
        return w_acc, i_acc

    w, i = lax.fori_loop(
        0, TOPK, body,
        (jnp.zeros((R, TOPK), jnp.float32), jnp.zeros((R, TOPK), jnp.int32)))
    w_ref[...] = w
    i_ref[...] = i


def _topk64_idx(x, norms, cols):
    B, W = x.shape
    return pl.pallas_call(
        _topk_idx_body,
        grid=(B // TQT,),
        in_specs=[pl.BlockSpec((TQT, W), lambda r: (r, 0)),
                  pl.BlockSpec((TQT, W), lambda r: (r, 0)),
                  pl.BlockSpec((TQT, W), lambda r: (r, 0))],
        out_specs=[
            pl.BlockSpec((TQT, TOPK), lambda r: (r, 0)),
            pl.BlockSpec((TQT, TOPK), lambda r: (r, 0)),
        ],
        out_shape=[
            jax.ShapeDtypeStruct((B, TOPK), jnp.float32),
            jax.ShapeDtypeStruct((B, TOPK), jnp.int32),
        ],
        scratch_shapes=[pltpu.VMEM((TQT, W), jnp.float32)],
        compiler_params=pltpu.CompilerParams(
            dimension_semantics=("parallel",)),
    )(x, norms, cols)


def _sc_gather(table, idx):
    info = plsc.get_sparse_core_info()
    nw = info.num_cores * info.num_subcores
    btot = idx.shape[0]
    bpw = btot // nw
    ch = 128
    mesh = plsc.VectorSubcoreMesh(core_axis_name="c", subcore_axis_name="s")

    @functools.partial(
        pl.kernel, mesh=mesh,
        out_type=jax.ShapeDtypeStruct((btot, D), jnp.float32),
        scratch_types=[
            pltpu.VMEM((ch,), jnp.int32),
            pltpu.VMEM((ch, D), jnp.float32),
            pltpu.SemaphoreType.DMA,
        ],
    )
    def k(table_hbm, idx_hbm, out_hbm, idx_v, rows_v, sem):
        wid = lax.axis_index("s") * info.num_cores + lax.axis_index("c")
        base = wid * bpw

        def body(i, carry):
            off = base + i * ch
            pltpu.sync_copy(idx_hbm.at[pl.ds(off, ch)], idx_v)
            pltpu.async_copy(table_hbm.at[idx_v], rows_v, sem).wait()
            pltpu.sync_copy(rows_v, out_hbm.at[pl.ds(off, ch)])
            return carry

        lax.fori_loop(0, bpw // ch, body, 0)

    return k(table, idx)


def _scale_body(v_ref, w_ref, o_ref):
    o_ref[...] = v_ref[...] * w_ref[...]


def _scale(vals, w):
    n = vals.shape[0]
    rt = 2048
    return pl.pallas_call(
        _scale_body,
        grid=(n // rt,),
        in_specs=[pl.BlockSpec((rt, D), lambda r: (r, 0)),
                  pl.BlockSpec((rt, 1), lambda r: (r, 0))],
        out_specs=pl.BlockSpec((rt, D), lambda r: (r, 0)),
        out_shape=jax.ShapeDtypeStruct((n, D), jnp.float32),
        compiler_params=pltpu.CompilerParams(
            dimension_semantics=("parallel",)),
    )(vals, w)


def kernel(query, key_dict, value_dict, top_n):
    B = query.shape[0]
    key_pad = jnp.pad(key_dict, ((0, NKP - NKEYS), (0, 0)))
    qn = jnp.sqrt(jnp.sum(query * query, axis=1, keepdims=True))
    kn = jnp.sqrt(jnp.sum(key_pad * key_pad, axis=1))[None, :]
    scores, l13, gm3 = _scores_and_group_max(query, key_pad, qn, kn)
    gm = gm3.transpose(1, 0, 2).reshape(B, NG2)
    l1m = l13.transpose(1, 0, 2).reshape(B, NG1)

    _, gi = _topk64(gm)
    cols1 = ((gi[:, :, None] // 8) * 128 + (gi[:, :, None] % 8) * 16
             + jnp.arange(16, dtype=gi.dtype)[None, None, :]).reshape(B, TOPK * 16)
    c1 = jnp.take_along_axis(l1m, cols1, axis=1)

    _, a1 = _topk64(c1)
    lid = jnp.take_along_axis(cols1, a1, axis=1)
    cols = ((lid[:, :, None] // G2) * KC + lid[:, :, None] % G2
            + G2 * jnp.arange(G1, dtype=lid.dtype)[None, None, :]
            ).reshape(B, TOPK * G1)
    cand = jnp.take_along_axis(scores, cols, axis=1)
    ncand = qn * kn.reshape(-1)[cols]

    w, idx = _topk64_idx(cand, ncand, cols)
    vals = _sc_gather(value_dict, idx.reshape(-1))
    return _scale(vals, w.reshape(-1, 1)).reshape(B, TOPK, D)


# speedup vs baseline: 5.9095x; 5.9095x over previous
import functools

import jax
import jax.numpy as jnp
from jax import lax
from jax.experimental import pallas as pl
from jax.experimental.pallas import tpu as pltpu
from jax.experimental.pallas import tpu_sc as plsc

NKEYS = 100000
D = 128
QT = 256
KC = 1024
G1 = 8
G2 = 128
NKP = 102400
NG1 = NKP // G1
NG2 = NKP // G2
TOPK = 64
TQT = 256


def _score_body(qn_ref, kn_ref, q_ref, k_ref, s_ref, l1_ref, gm_ref):
    kc = pl.program_id(1)
    q = q_ref[...]
    k = k_ref[...]
    s = lax.dot_general(q, k, (((1,), (1,)), ((), ())),
                        preferred_element_type=jnp.float32)
    norm = qn_ref[...] * kn_ref[...]
    norm = jnp.where(norm > 1e-8, norm, 1e-8)
    cos = s / norm
    col = kc * KC + lax.broadcasted_iota(jnp.int32, (QT, KC), 1)
    cos = jnp.where(col < NKEYS, cos, -1e30)
    s_ref[...] = cos
    l1 = cos[:, 0:G2]
    for j in range(1, KC // G2):
        l1 = jnp.maximum(l1, cos[:, j * G2:(j + 1) * G2])
    l1_ref[0] = l1
    m = l1
    for sh in (8, 4, 2, 1):
        m = jnp.maximum(m, pltpu.roll(m, G2 - sh, 1))
    nh = KC // G2
    onehot = (lax.broadcasted_iota(jnp.int32, (G2, nh), 0)
              == 16 * lax.broadcasted_iota(jnp.int32, (G2, nh), 1)
              ).astype(jnp.float32)
    gm_ref[0] = lax.dot_general(m, onehot, (((1,), (0,)), ((), ())),
                                precision=lax.Precision.HIGHEST,
                                preferred_element_type=jnp.float32)


def _scores_and_group_max(query, key_pad, qn, kn):
    B = query.shape[0]
    return pl.pallas_call(
        _score_body,
        grid=(B // QT, NKP // KC),
        in_specs=[
            pl.BlockSpec((QT, 1), lambda q, c: (q, 0)),
            pl.BlockSpec((1, KC), lambda q, c: (0, c)),
            pl.BlockSpec((QT, D), lambda q, c: (q, 0)),
            pl.BlockSpec((KC, D), lambda q, c: (c, 0)),
        ],
        out_specs=[
            pl.BlockSpec((QT, KC), lambda q, c: (q, c)),
            pl.BlockSpec((1, QT, G2), lambda q, c: (c, q, 0)),
            pl.BlockSpec((1, QT, KC // G2), lambda q, c: (c, q, 0)),
        ],
        out_shape=[
            jax.ShapeDtypeStruct((B, NKP), jnp.float32),
            jax.ShapeDtypeStruct((NKP // KC, B, G2), jnp.float32),
            jax.ShapeDtypeStruct((NKP // KC, B, KC // G2), jnp.float32),
        ],
        compiler_params=pltpu.CompilerParams(
            dimension_semantics=("parallel", "arbitrary")),
    )(qn, kn, query, key_pad)


def _topk_body(x_ref, w_ref, i_ref, xs_ref):
    R, W = x_ref.shape
    xs_ref[...] = x_ref[...]
    ii = lax.broadcasted_iota(jnp.int32, (R, W), 1)
    t64 = lax.broadcasted_iota(jnp.int32, (R, TOPK), 1)

    def body(t, carry):
        w_acc, i_acc = carry
        x = xs_ref[...]
        m = jnp.max(x, axis=1, keepdims=True)
        am = jnp.min(jnp.where(x == m, ii, W), axis=1, keepdims=True)
        xs_ref[...] = jnp.where(ii == am, -jnp.inf, x)
        w_acc = jnp.where(t64 == t, m, w_acc)
        i_acc = jnp.where(t64 == t, am, i_acc)
        return w_acc, i_acc

    w, i = lax.fori_loop(
        0, TOPK, body,
        (jnp.zeros((R, TOPK), jnp.float32), jnp.zeros((R, TOPK), jnp.int32)))
    w_ref[...] = w
    i_ref[...] = i


def _topk64(x):
    B, W = x.shape
    return pl.pallas_call(
        _topk_body,
        grid=(B // TQT,),
        in_specs=[pl.BlockSpec((TQT, W), lambda r: (r, 0))],
        out_specs=[
            pl.BlockSpec((TQT, TOPK), lambda r: (r, 0)),
            pl.BlockSpec((TQT, TOPK), lambda r: (r, 0)),
        ],
        out_shape=[
            jax.ShapeDtypeStruct((B, TOPK), jnp.float32),
            jax.ShapeDtypeStruct((B, TOPK), jnp.int32),
        ],
        scratch_shapes=[pltpu.VMEM((TQT, W), jnp.float32)],
        compiler_params=pltpu.CompilerParams(
            dimension_semantics=("parallel",)),
    )(x)


def _topk_idx_body(x_ref, c_ref, w_ref, i_ref, xs_ref):
    R, W = x_ref.shape
    xs_ref[...] = x_ref[...]
    cmap = c_ref[...]
    t64 = lax.broadcasted_iota(jnp.int32, (R, TOPK), 1)

    def body(t, carry):
        w_acc, i_acc = carry
        x = xs_ref[...]
        m = jnp.max(x, axis=1, keepdims=True)
        am = jnp.min(jnp.where(x == m, cmap, 2 ** 31 - 1), axis=1,
                     keepdims=True)
        xs_ref[...] = jnp.where(cmap == am, -jnp.inf, x)
        w_acc = jnp.where(t64 == t, m, w_acc)
        i_acc = jnp.where(t64 == t, am, i_acc)
        return w_acc, i_acc

    w, i = lax.fori_loop(
        0, TOPK, body,
        (jnp.zeros((R, TOPK), jnp.float32), jnp.zeros((R, TOPK), jnp.int32)))
    w_ref[...] = w
    i_ref[...] = i


def _topk64_idx(x, cols):
    B, W = x.shape
    return pl.pallas_call(
        _topk_idx_body,
        grid=(B // TQT,),
        in_specs=[pl.BlockSpec((TQT, W), lambda r: (r, 0)),
                  pl.BlockSpec((TQT, W), lambda r: (r, 0))],
        out_specs=[
            pl.BlockSpec((TQT, TOPK), lambda r: (r, 0)),
            pl.BlockSpec((TQT, TOPK), lambda r: (r, 0)),
        ],
        out_shape=[
            jax.ShapeDtypeStruct((B, TOPK), jnp.float32),
            jax.ShapeDtypeStruct((B, TOPK), jnp.int32),
        ],
        scratch_shapes=[pltpu.VMEM((TQT, W), jnp.float32)],
        compiler_params=pltpu.CompilerParams(
            dimension_semantics=("parallel",)),
    )(x, cols)


def _sc_gather(table, idx):
    info = plsc.get_sparse_core_info()
    nw = info.num_cores * info.num_subcores
    btot = idx.shape[0]
    bpw = btot // nw
    ch = 128
    mesh = plsc.VectorSubcoreMesh(core_axis_name="c", subcore_axis_name="s")

    @functools.partial(
        pl.kernel, mesh=mesh,
        out_type=jax.ShapeDtypeStruct((btot, D), jnp.float32),
        scratch_types=[
            pltpu.VMEM((ch,), jnp.int32),
            pltpu.VMEM((ch, D), jnp.float32),
            pltpu.SemaphoreType.DMA,
        ],
    )
    def k(table_hbm, idx_hbm, out_hbm, idx_v, rows_v, sem):
        wid = lax.axis_index("s") * info.num_cores + lax.axis_index("c")
        base = wid * bpw

        def body(i, carry):
            off = base + i * ch
            pltpu.sync_copy(idx_hbm.at[pl.ds(off, ch)], idx_v)
            pltpu.async_copy(table_hbm.at[idx_v], rows_v, sem).wait()
            pltpu.sync_copy(rows_v, out_hbm.at[pl.ds(off, ch)])
            return carry

        lax.fori_loop(0, bpw // ch, body, 0)

    return k(table, idx)


def _scale_body(v_ref, w_ref, o_ref):
    o_ref[...] = v_ref[...] * w_ref[...]


def _scale(vals, w):
    n = vals.shape[0]
    rt = 2048
    return pl.pallas_call(
        _scale_body,
        grid=(n // rt,),
        in_specs=[pl.BlockSpec((rt, D), lambda r: (r, 0)),
                  pl.BlockSpec((rt, 1), lambda r: (r, 0))],
        out_specs=pl.BlockSpec((rt, D), lambda r: (r, 0)),
        out_shape=jax.ShapeDtypeStruct((n, D), jnp.float32),
        compiler_params=pltpu.CompilerParams(
            dimension_semantics=("parallel",)),
    )(vals, w)


def kernel(query, key_dict, value_dict, top_n):
    B = query.shape[0]
    key_pad = jnp.pad(key_dict, ((0, NKP - NKEYS), (0, 0)))
    qn = jnp.sqrt(jnp.sum(query * query, axis=1, keepdims=True))
    kn = jnp.sqrt(jnp.sum(key_pad * key_pad, axis=1))[None, :]
    scores, l13, gm3 = _scores_and_group_max(query, key_pad, qn, kn)
    gm = gm3.transpose(1, 0, 2).reshape(B, NG2)
    l1m = l13.transpose(1, 0, 2).reshape(B, NG1)

    _, gi = _topk64(gm)
    cols1 = ((gi[:, :, None] // 8) * 128 + (gi[:, :, None] % 8) * 16
             + jnp.arange(16, dtype=gi.dtype)[None, None, :]).reshape(B, TOPK * 16)
    c1 = jnp.take_along_axis(l1m, cols1, axis=1)

    _, a1 = _topk64(c1)
    lid = jnp.take_along_axis(cols1, a1, axis=1)
    cols = ((lid[:, :, None] // G2) * KC + lid[:, :, None] % G2
            + G2 * jnp.arange(G1, dtype=lid.dtype)[None, None, :]
            ).reshape(B, TOPK * G1)
    cand = jnp.take_along_axis(scores, cols, axis=1)

    w, idx = _topk64_idx(cand, cols)
    vals = _sc_gather(value_dict, idx.reshape(-1))
    return _scale(vals, w.reshape(-1, 1)).reshape(B, TOPK, D)
